# baseline (speedup 1.0000x reference)
import functools

import jax
import jax.numpy as jnp
from jax import lax
from jax.experimental import pallas as pl
from jax.experimental.pallas import tpu as pltpu
from jax.experimental.pallas import tpu_sc as plsc

NE = 8192
ED = 256
NT = 16384
BT = 512
BN = 1024
NB = NT // BT
NCHUNK = NE // BN
COMMIT = 0.1


_BOUNDS = (0, 2736, 5472, 8192)


_BIG = 2**30
_LG = 128


def _dist_body(x_ref, w_ref, sx_ref, sw_ref, idx_ref, minsum_ref):
    xb = x_ref[...].astype(jnp.bfloat16)
    sx = sx_ref[...]

    r_cmp = jnp.full((BT,), jnp.inf, jnp.float32)
    v_sel = jnp.full((BT,), jnp.inf, jnp.float32)
    i_sel = jnp.zeros((BT,), jnp.int32)
    for k in range(3):
        lo, hi = _BOUNDS[k], _BOUNDS[k + 1]
        wb = w_ref[lo:hi, :]
        mm = lax.dot_general(xb, wb, (((1,), (1,)), ((), ())),
                             preferred_element_type=jnp.float32)
        d = (sx[:, None] + sw_ref[lo:hi][None, :]) - 2.0 * mm
        cmin = jnp.min(d, axis=1)
        iota = lax.broadcasted_iota(jnp.int32, (BT, hi - lo), 1)
        cidx = jnp.min(jnp.where(d == cmin[:, None], iota, _BIG),
                       axis=1) + lo
        upd = cmin < r_cmp
        v_sel = jnp.where(upd, cmin, v_sel)
        i_sel = jnp.where(upd, cidx, i_sel)
        r_cmp = jnp.where(upd, cmin, r_cmp).astype(jnp.bfloat16).astype(jnp.float32)
    idx_ref[...] = i_sel
    minsum_ref[0, 0, 0] = jnp.sum(v_sel)


_dist_call = pl.pallas_call(
    _dist_body,
    grid=(NB,),
    in_specs=[
        pl.BlockSpec((BT, ED), lambda i: (i, 0)),
        pl.BlockSpec((NE, ED), lambda i: (0, 0)),
        pl.BlockSpec((BT,), lambda i: (i,)),
        pl.BlockSpec((NE,), lambda i: (0,)),
    ],
    out_specs=[
        pl.BlockSpec((BT,), lambda i: (i,)),
        pl.BlockSpec((1, 1, 1), lambda i: (i, 0, 0), memory_space=pltpu.SMEM),
    ],
    out_shape=[
        jax.ShapeDtypeStruct((NT,), jnp.int32),
        jax.ShapeDtypeStruct((NB, 1, 1), jnp.float32),
    ],
)


_NC, _NS = 2, 16
_NW = _NC * _NS
_BPW = NT // _NW
_CH = 128
_NCH = _BPW // _CH


def _gather_fn(table_hbm, idx_hbm, out_hbm, idx_v, buf, sem):
    wid = lax.axis_index("s") * _NC + lax.axis_index("c")
    base = wid * _BPW
    pltpu.sync_copy(idx_hbm.at[pl.ds(base, _BPW)], idx_v)
    for c in range(_NCH):
        pltpu.async_copy(table_hbm.at[idx_v.at[pl.ds(c * _CH, _CH)]],
                         buf, sem).wait()
        pltpu.sync_copy(buf, out_hbm.at[pl.ds(base + c * _CH, _CH)])


@functools.cache
def _gather_call():
    return functools.partial(
        pl.kernel,
        out_type=jax.ShapeDtypeStruct((NT, ED), jnp.float32),
        mesh=plsc.VectorSubcoreMesh(core_axis_name="c", subcore_axis_name="s"),
        scratch_types=[
            pltpu.VMEM((_BPW,), jnp.int32),
            pltpu.VMEM((_CH, ED), jnp.float32),
            pltpu.SemaphoreType.DMA,
        ],
    )(_gather_fn)


def kernel(x, W):
    flat_x = x.reshape(-1, ED)
    sx = jnp.sum(flat_x ** 2, axis=1)
    sw = jnp.sum(W ** 2, axis=1)
    idx, minsum = _dist_call(flat_x, W.astype(jnp.bfloat16), sx, sw)
    q = _gather_call()(W, idx)
    m = jnp.sum(minsum) / (NT * ED)
    loss = m + COMMIT * m
    return (q.reshape(x.shape), loss, idx[:, None])


# speedup vs baseline: 1.4341x; 1.4341x over previous
import functools

import jax
import jax.numpy as jnp
from jax import lax
from jax.experimental import pallas as pl
from jax.experimental.pallas import tpu as pltpu
from jax.experimental.pallas import tpu_sc as plsc

NE = 8192
ED = 256
NT = 16384
BT = 512
BN = 1024
NB = NT // BT
NCHUNK = NE // BN
COMMIT = 0.1


_BOUNDS = (0, 2736, 5472, 8192)


_BIG = 2**30
_LG = 128


def _dist_body(x_ref, w_ref, sx_ref, sw_ref, idx_ref, minsum_ref):
    xb = x_ref[...].astype(jnp.bfloat16)
    sx = sx_ref[...]

    r_cmp = jnp.full((BT,), jnp.inf, jnp.float32)
    v_sel = jnp.full((BT,), jnp.inf, jnp.float32)
    i_sel = jnp.zeros((BT,), jnp.int32)
    for k in range(3):
        lo, hi = _BOUNDS[k], _BOUNDS[k + 1]
        wb = w_ref[lo:hi, :]
        mm = lax.dot_general(xb, wb, (((1,), (1,)), ((), ())),
                             preferred_element_type=jnp.float32)
        d = (sx[:, None] + sw_ref[lo:hi][None, :]) - 2.0 * mm
        cmin = jnp.min(d, axis=1)
        iota = lax.broadcasted_iota(jnp.int32, (BT, hi - lo), 1)
        cidx = jnp.min(jnp.where(d == cmin[:, None], iota, _BIG),
                       axis=1) + lo
        upd = cmin < r_cmp
        v_sel = jnp.where(upd, cmin, v_sel)
        i_sel = jnp.where(upd, cidx, i_sel)
        r_cmp = jnp.where(upd, cmin, r_cmp).astype(jnp.bfloat16).astype(jnp.float32)
    idx_ref[...] = i_sel
    minsum_ref[0, 0, 0] = jnp.sum(v_sel)


_dist_call = pl.pallas_call(
    _dist_body,
    grid=(NB,),
    in_specs=[
        pl.BlockSpec((BT, ED), lambda i: (i, 0)),
        pl.BlockSpec((NE, ED), lambda i: (0, 0)),
        pl.BlockSpec((BT,), lambda i: (i,)),
        pl.BlockSpec((NE,), lambda i: (0,)),
    ],
    out_specs=[
        pl.BlockSpec((BT,), lambda i: (i,)),
        pl.BlockSpec((1, 1, 1), lambda i: (i, 0, 0), memory_space=pltpu.SMEM),
    ],
    out_shape=[
        jax.ShapeDtypeStruct((NT,), jnp.int32),
        jax.ShapeDtypeStruct((NB, 1, 1), jnp.float32),
    ],
)


_NC, _NS = 2, 16
_NW = _NC * _NS
_BPW = NT // _NW
_CH = 128
_NCH = _BPW // _CH


def _gather_fn(table_hbm, idx_hbm, out_hbm, idx_v, buf0, buf1, sem0, sem1):
    wid = lax.axis_index("s") * _NC + lax.axis_index("c")
    base = wid * _BPW
    pltpu.sync_copy(idx_hbm.at[pl.ds(base, _BPW)], idx_v)
    bufs, sems = (buf0, buf1), (sem0, sem1)
    pend = pltpu.async_copy(table_hbm.at[idx_v.at[pl.ds(0, _CH)]],
                            bufs[0], sems[0])
    for c in range(_NCH):
        cur = pend
        if c + 1 < _NCH:
            j = (c + 1) % 2
            pend = pltpu.async_copy(
                table_hbm.at[idx_v.at[pl.ds((c + 1) * _CH, _CH)]],
                bufs[j], sems[j])
        cur.wait()
        pltpu.sync_copy(bufs[c % 2], out_hbm.at[pl.ds(base + c * _CH, _CH)])


@functools.cache
def _gather_call():
    return functools.partial(
        pl.kernel,
        out_type=jax.ShapeDtypeStruct((NT, ED), jnp.float32),
        mesh=plsc.VectorSubcoreMesh(core_axis_name="c", subcore_axis_name="s"),
        scratch_types=[
            pltpu.VMEM((_BPW,), jnp.int32),
            pltpu.VMEM((_CH, ED), jnp.float32),
            pltpu.VMEM((_CH, ED), jnp.float32),
            pltpu.SemaphoreType.DMA,
            pltpu.SemaphoreType.DMA,
        ],
    )(_gather_fn)


def kernel(x, W):
    flat_x = x.reshape(-1, ED)
    sx = jnp.sum(flat_x ** 2, axis=1)
    sw = jnp.sum(W ** 2, axis=1)
    idx, minsum = _dist_call(flat_x, W.astype(jnp.bfloat16), sx, sw)
    q = _gather_call()(W, idx)
    m = jnp.sum(minsum) / (NT * ED)
    loss = m + COMMIT * m
    return (q.reshape(x.shape), loss, idx[:, None])


# speedup vs baseline: 1.5417x; 1.0751x over previous
import functools

import jax
import jax.numpy as jnp
from jax import lax
from jax.experimental import pallas as pl
from jax.experimental.pallas import tpu as pltpu
from jax.experimental.pallas import tpu_sc as plsc

NE = 8192
ED = 256
NT = 16384
BT = 512
BN = 1024
NB = NT // BT
NCHUNK = NE // BN
COMMIT = 0.1


_BOUNDS = (0, 2736, 5472, 8192)
_NPAD = 2816


_BIG = 2**30


def _dist_body(x_ref, w_ref, sx_ref, sw_ref, idx_ref, minsum_ref):
    xb = x_ref[...].astype(jnp.bfloat16)
    sx = sx_ref[...]

    r_cmp = jnp.full((BT,), jnp.inf, jnp.float32)
    v_sel = jnp.full((BT,), jnp.inf, jnp.float32)
    i_sel = jnp.zeros((BT,), jnp.float32)
    for k in range(3):
        lo = k * _NPAD
        wb = w_ref[lo:lo + _NPAD, :]
        mm2 = lax.dot_general(xb, wb, (((1,), (1,)), ((), ())),
                              preferred_element_type=jnp.float32)
        d = (sx[:, None] + sw_ref[lo:lo + _NPAD][None, :]) + mm2
        cmin = jnp.min(d, axis=1)
        iota = lax.broadcasted_iota(jnp.int32, (BT, _NPAD), 1).astype(jnp.float32)
        cidx = jnp.min(jnp.where(d == cmin[:, None], iota, jnp.float32(1e9)),
                       axis=1) + _BOUNDS[k]
        upd = cmin < r_cmp
        v_sel = jnp.where(upd, cmin, v_sel)
        i_sel = jnp.where(upd, cidx, i_sel)
        r_cmp = jnp.where(upd, cmin, r_cmp).astype(jnp.bfloat16).astype(jnp.float32)
    idx_ref[...] = i_sel.astype(jnp.int32)
    minsum_ref[0, 0, 0] = jnp.sum(v_sel)


_dist_call = pl.pallas_call(
    _dist_body,
    grid=(NB,),
    in_specs=[
        pl.BlockSpec((BT, ED), lambda i: (i, 0)),
        pl.BlockSpec((3 * _NPAD, ED), lambda i: (0, 0)),
        pl.BlockSpec((BT,), lambda i: (i,)),
        pl.BlockSpec((3 * _NPAD,), lambda i: (0,)),
    ],
    out_specs=[
        pl.BlockSpec((BT,), lambda i: (i,)),
        pl.BlockSpec((1, 1, 1), lambda i: (i, 0, 0), memory_space=pltpu.SMEM),
    ],
    out_shape=[
        jax.ShapeDtypeStruct((NT,), jnp.int32),
        jax.ShapeDtypeStruct((NB, 1, 1), jnp.float32),
    ],
)


_NC, _NS = 2, 16
_NW = _NC * _NS
_BPW = NT // _NW
_CH = 128
_NCH = _BPW // _CH


def _gather_fn(table_hbm, idx_hbm, out_hbm, idx_v, buf0, buf1, sem0, sem1):
    wid = lax.axis_index("s") * _NC + lax.axis_index("c")
    base = wid * _BPW
    pltpu.sync_copy(idx_hbm.at[pl.ds(base, _BPW)], idx_v)
    bufs, sems = (buf0, buf1), (sem0, sem1)
    pend = pltpu.async_copy(table_hbm.at[idx_v.at[pl.ds(0, _CH)]],
                            bufs[0], sems[0])
    for c in range(_NCH):
        cur = pend
        if c + 1 < _NCH:
            j = (c + 1) % 2
            pend = pltpu.async_copy(
                table_hbm.at[idx_v.at[pl.ds((c + 1) * _CH, _CH)]],
                bufs[j], sems[j])
        cur.wait()
        pltpu.sync_copy(bufs[c % 2], out_hbm.at[pl.ds(base + c * _CH, _CH)])


@functools.cache
def _gather_call():
    return functools.partial(
        pl.kernel,
        out_type=jax.ShapeDtypeStruct((NT, ED), jnp.float32),
        mesh=plsc.VectorSubcoreMesh(core_axis_name="c", subcore_axis_name="s"),
        scratch_types=[
            pltpu.VMEM((_BPW,), jnp.int32),
            pltpu.VMEM((_CH, ED), jnp.float32),
            pltpu.VMEM((_CH, ED), jnp.float32),
            pltpu.SemaphoreType.DMA,
            pltpu.SemaphoreType.DMA,
        ],
    )(_gather_fn)


def _pad_passes(a, fill):
    parts = []
    for k in range(3):
        lo, hi = _BOUNDS[k], _BOUNDS[k + 1]
        seg = a[lo:hi]
        pad = jnp.full((_NPAD - (hi - lo),) + a.shape[1:], fill, a.dtype)
        parts += [seg, pad]
    return jnp.concatenate(parts, axis=0)


def kernel(x, W):
    flat_x = x.reshape(-1, ED)
    sx = jnp.sum(flat_x ** 2, axis=1)
    sw = jnp.sum(W ** 2, axis=1)
    w_pad = _pad_passes((-2.0 * W).astype(jnp.bfloat16), 0)
    sw_pad = _pad_passes(sw, jnp.inf)
    idx, minsum = _dist_call(flat_x, w_pad, sx, sw_pad)
    q = _gather_call()(W, idx)
    m = jnp.sum(minsum) / (NT * ED)
    loss = m + COMMIT * m
    return (q.reshape(x.shape), loss, idx[:, None])
